# Initial kernel scaffold; baseline (speedup 1.0000x reference)
import jax
import jax.numpy as jnp
from jax.experimental import pallas as pl


def kernel(get_train, label, gender_idx, ucc_idx, ucs_idx, lcc_idx, lcs_idx, hat_idx, backpack_idx, token_prefix, token_suffix, gender_ctx, ucc_ctx, ucs_ctx, lcc_ctx, lcs_ctx, hat_ctx, backpack_ctx, cls_ctx):
    raise NotImplementedError("write your pallas kernel here")


# speedup vs baseline: 1.0373x; 1.0373x over previous
import functools

import jax
import jax.numpy as jnp
from jax import lax
from jax.experimental import pallas as pl
from jax.experimental.pallas import tpu as pltpu
from jax.experimental.pallas import tpu_sc as plsc

B = 1024
CTX_DIM = 512
PREFIX_LEN = 5
SUFFIX_LEN = 40
N_CLS = 4
N_ATTR_SLOTS = 28
SMALL_ROWS = 176
SEQ = PREFIX_LEN + N_CLS + N_ATTR_SLOTS + SUFFIX_LEN

NB = 8
N_WORKERS = 32
B_PER_W = B // N_WORKERS


def _sc_cls_gather(label, cls_ctx):
    mesh = plsc.VectorSubcoreMesh(core_axis_name="c", subcore_axis_name="s")

    @functools.partial(
        pl.kernel,
        out_type=jax.ShapeDtypeStruct((B, N_CLS, CTX_DIM), jnp.float32),
        mesh=mesh,
        scratch_types=[
            pltpu.VMEM((B_PER_W,), jnp.int32),
            pltpu.VMEM((B_PER_W, N_CLS, CTX_DIM), jnp.float32),
            pltpu.SemaphoreType.DMA,
        ],
    )
    def gather_kernel(label_hbm, cls_hbm, out_hbm, idx_v, rows_v, sem):
        wid = lax.axis_index("s") * 2 + lax.axis_index("c")
        base = wid * B_PER_W
        pltpu.sync_copy(label_hbm.at[pl.ds(base, B_PER_W)], idx_v)
        pltpu.async_copy(cls_hbm.at[idx_v], rows_v, sem).wait()
        pltpu.sync_copy(rows_v, out_hbm.at[pl.ds(base, B_PER_W)])

    return gather_kernel(label, cls_ctx)


def _tc_assemble_body(cls_ref, pre_ref, suf_ref, hi_ref, lo_ref, sel_ref,
                      out_ref):
    out_ref[:, 0:PREFIX_LEN, :] = jnp.broadcast_to(
        pre_ref[...][None], (NB, PREFIX_LEN, CTX_DIM))
    out_ref[:, PREFIX_LEN:PREFIX_LEN + N_CLS, :] = cls_ref[...]
    out_ref[:, SEQ - SUFFIX_LEN:SEQ, :] = jnp.broadcast_to(
        suf_ref[...][None], (NB, SUFFIX_LEN, CTX_DIM))
    sel = sel_ref[...]
    onehot = (sel == lax.broadcasted_iota(jnp.int32, (1, SMALL_ROWS), 1))
    onehot = onehot.astype(jnp.bfloat16)
    res = (jnp.dot(onehot, hi_ref[...], preferred_element_type=jnp.float32)
           + jnp.dot(onehot, lo_ref[...], preferred_element_type=jnp.float32))
    a0 = PREFIX_LEN + N_CLS
    for b in range(NB):
        out_ref[b, a0:a0 + N_ATTR_SLOTS, :] = (
            res[b * N_ATTR_SLOTS:(b + 1) * N_ATTR_SLOTS, :])


def _tc_assemble(cls_part, prefix, suffix, small_hi, small_lo, attr_sel):
    return pl.pallas_call(
        _tc_assemble_body,
        grid=(B // NB,),
        in_specs=[
            pl.BlockSpec((NB, N_CLS, CTX_DIM), lambda i: (i, 0, 0)),
            pl.BlockSpec((PREFIX_LEN, CTX_DIM), lambda i: (0, 0)),
            pl.BlockSpec((SUFFIX_LEN, CTX_DIM), lambda i: (0, 0)),
            pl.BlockSpec((SMALL_ROWS, CTX_DIM), lambda i: (0, 0)),
            pl.BlockSpec((SMALL_ROWS, CTX_DIM), lambda i: (0, 0)),
            pl.BlockSpec((NB * N_ATTR_SLOTS, 1), lambda i: (i, 0)),
        ],
        out_specs=pl.BlockSpec((NB, SEQ, CTX_DIM), lambda i: (i, 0, 0)),
        out_shape=jax.ShapeDtypeStruct((B, SEQ, CTX_DIM), jnp.float32),
    )(cls_part, prefix, suffix, small_hi, small_lo, attr_sel)


def kernel(get_train, label, gender_idx, ucc_idx, ucs_idx, lcc_idx, lcs_idx,
           hat_idx, backpack_idx, token_prefix, token_suffix, gender_ctx,
           ucc_ctx, ucs_ctx, lcc_ctx, lcs_ctx, hat_ctx, backpack_ctx,
           cls_ctx):
    del get_train
    tables = (gender_ctx, ucc_ctx, ucs_ctx, lcc_ctx, lcs_ctx, hat_ctx,
              backpack_ctx)
    idxs = (gender_idx, ucc_idx, ucs_idx, lcc_idx, lcs_idx, hat_idx,
            backpack_idx)
    small = jnp.concatenate([t.reshape(-1, CTX_DIM) for t in tables], axis=0)
    small_hi = small.astype(jnp.bfloat16)
    small_lo = (small - small_hi.astype(jnp.float32)).astype(jnp.bfloat16)

    sel_parts = []
    base = 0
    j = jnp.arange(4, dtype=jnp.int32)[None, :]
    for t, ix in zip(tables, idxs):
        sel_parts.append(base + 4 * ix.astype(jnp.int32)[:, None] + j)
        base += t.shape[0] * 4
    attr_sel = jnp.concatenate(sel_parts, axis=1).reshape(
        B * N_ATTR_SLOTS, 1)

    prefix = token_prefix.reshape(PREFIX_LEN, CTX_DIM)
    suffix = token_suffix.reshape(SUFFIX_LEN, CTX_DIM)

    cls_part = _sc_cls_gather(label.astype(jnp.int32), cls_ctx)

    return _tc_assemble(cls_part, prefix, suffix, small_hi, small_lo,
                        attr_sel)
